# baseline (speedup 1.0000x reference)
import functools

import jax
import jax.numpy as jnp
from jax import lax
from jax.experimental import pallas as pl
from jax.experimental.pallas import tpu as pltpu
from jax.experimental.pallas import tpu_sc as plsc

_COL = 1024
_ROW = 512
_LAYERS = 3
_EK = 16384
_EG = 65536

_NC = 2
_NS = 16
_NW = _NC * _NS

_ZB = 4096


_SHK = (_ROW // _NC) * _ROW
_SHG = (_COL // _NC) * _COL
_TRASH = 128


def _sc_count_body(kei, gei, zhb, ohb, out_k, out_g, sh_k, sh_g,
                   zbuf, ones_v, idx_a, idx_b, idx_c, idx_d,
                   dvm_k, svm_k, dvm_g, svm_g,
                   sem_e, sem_z, sem_a, sem_b, sem_c, sem_d):
    c = lax.axis_index("c")
    s = lax.axis_index("s")

    ek = _EK // _NS
    eg = _EG // _NS
    loads = [
        pltpu.async_copy(kei.at[1, pl.ds(s * ek, ek)], dvm_k, sem_e),
        pltpu.async_copy(kei.at[0, pl.ds(s * ek, ek)], svm_k, sem_e),
        pltpu.async_copy(gei.at[1, pl.ds(s * eg, eg)], dvm_g, sem_e),
        pltpu.async_copy(gei.at[0, pl.ds(s * eg, eg)], svm_g, sem_e),
        pltpu.async_copy(ohb, ones_v, sem_e),
    ]
    pltpu.async_copy(zhb, zbuf, sem_z).wait()

    ksl = _SHK // _NS
    gsl = _SHG // _NS
    zeros = []
    for r in range(ksl // _ZB):
        zeros.append(pltpu.async_copy(
            zbuf, sh_k.at[pl.ds(s * ksl + r * _ZB, _ZB)], sem_z))
    for r in range(gsl // _ZB):
        zeros.append(pltpu.async_copy(
            zbuf, sh_g.at[pl.ds(s * gsl + r * _ZB, _ZB)], sem_z))
    for h in loads:
        h.wait()
    for h in zeros:
        h.wait()
    plsc.subcore_barrier()

    def scatter_graph(rvm, cvm, shared, n, e_tile, trash, blk_sh):
        r0 = c * (n // _NC)
        bufs = (idx_a, idx_b, idx_c, idx_d)
        sems = (sem_a, sem_b, sem_c, sem_d)
        handles = [None, None, None, None]
        for chunk in range(e_tile // 128):
            p = chunk % 4
            if handles[p] is not None:
                handles[p].wait()
            for j in range(8):
                off = chunk * 128 + j * 16
                rl = rvm[pl.ds(off, 16)] - r0
                cc = cvm[pl.ds(off, 16)]
                ok = (rl >= 0) & (rl < (n // _NC))
                flat = ((cc >> 7) << blk_sh) | (rl << 7) | (cc & 127)
                flat = jnp.where(ok, flat, trash + (cc & 127))
                bufs[p][pl.ds(j * 16, 16)] = flat
            handles[p] = pltpu.async_copy(
                ones_v, shared.at[bufs[p]], sems[p], add=True)
        for h in handles:
            if h is not None:
                h.wait()

    scatter_graph(svm_k, dvm_k, sh_k, _ROW, ek, _SHK, 15)
    scatter_graph(dvm_g, svm_g, sh_g, _COL, eg, _SHG, 16)
    plsc.subcore_barrier()

    outs = []
    kb = _SHK // 4
    gb = _SHG // 8
    for j in range(4):
        outs.append(pltpu.async_copy(
            sh_k.at[pl.ds(j * kb + s * (kb // _NS), kb // _NS)],
            out_k.at[pl.ds(j * 2 * kb + c * kb + s * (kb // _NS), kb // _NS)],
            sem_a))
    for j in range(8):
        outs.append(pltpu.async_copy(
            sh_g.at[pl.ds(j * gb + s * (gb // _NS), gb // _NS)],
            out_g.at[pl.ds(j * 2 * gb + c * gb + s * (gb // _NS), gb // _NS)],
            sem_b))
    for h in outs:
        h.wait()


def _sc_counts(kei, gei):
    mesh = plsc.VectorSubcoreMesh(core_axis_name="c", subcore_axis_name="s")
    f32 = jnp.float32
    i32 = jnp.int32
    run = functools.partial(
        pl.kernel,
        mesh=mesh,
        out_type=[
            jax.ShapeDtypeStruct((_ROW * _ROW,), f32),
            jax.ShapeDtypeStruct((_COL * _COL,), f32),
        ],
        scratch_types=[
            pltpu.VMEM_SHARED((_SHK + _TRASH,), f32),
            pltpu.VMEM_SHARED((_SHG + _TRASH,), f32),
            pltpu.VMEM((_ZB,), f32),
            pltpu.VMEM((128,), f32),
            pltpu.VMEM((128,), i32),
            pltpu.VMEM((128,), i32),
            pltpu.VMEM((128,), i32),
            pltpu.VMEM((128,), i32),
            pltpu.VMEM((_EK // _NS,), i32),
            pltpu.VMEM((_EK // _NS,), i32),
            pltpu.VMEM((_EG // _NS,), i32),
            pltpu.VMEM((_EG // _NS,), i32),
            pltpu.SemaphoreType.DMA,
            pltpu.SemaphoreType.DMA,
            pltpu.SemaphoreType.DMA,
            pltpu.SemaphoreType.DMA,
            pltpu.SemaphoreType.DMA,
            pltpu.SemaphoreType.DMA,
        ],
    )(_sc_count_body)
    zhb = jnp.zeros((_ZB,), f32)
    ohb = jnp.ones((128,), f32)
    return run(kei, gei, zhb, ohb)


def _conv_body(wlc_ref, wrc_ref, wlr_ref, wrr_ref,
               owlc_ref, owrc_ref, owlrT_ref, owrrT_ref):
    bf16 = jnp.bfloat16
    owlc_ref[...] = wlc_ref[...].astype(bf16)
    owrc_ref[...] = wrc_ref[...].astype(bf16)
    owlrT_ref[0] = wlr_ref[0].astype(bf16).T
    owrrT_ref[0] = wrr_ref[0].astype(bf16).T


def _convert_weights(Wl_c, Wr_c, Wl_r, Wr_r):
    bf16 = jnp.bfloat16
    bc = lambda shape: [pl.BlockSpec((1,) + shape, lambda i: (i, 0, 0))]
    return pl.pallas_call(
        _conv_body,
        grid=(_LAYERS,),
        in_specs=(bc((_COL, _COL)) * 2 + bc((_ROW, _ROW)) * 2),
        out_specs=(bc((_COL, _COL)) * 2 + bc((_ROW, _ROW)) * 2),
        out_shape=[
            jax.ShapeDtypeStruct((_LAYERS, _COL, _COL), bf16),
            jax.ShapeDtypeStruct((_LAYERS, _COL, _COL), bf16),
            jax.ShapeDtypeStruct((_LAYERS, _ROW, _ROW), bf16),
            jax.ShapeDtypeStruct((_LAYERS, _ROW, _ROW), bf16),
        ],
        compiler_params=pltpu.CompilerParams(
            dimension_semantics=("arbitrary",),
        ),
    )(Wl_c, Wr_c, Wl_r, Wr_r)


def _tc_net_body(ctk_ref, cg_ref, x_ref, wlc_ref, wrc_ref, bc_ref,
                 wlrT_ref, wrrT_ref, br_ref, out_ref,
                 e_s, ckb_s, cgb_s, rck_s, rcg_s):
    i = pl.program_id(0)
    f32 = jnp.float32
    bf16 = jnp.bfloat16

    @pl.when(i == 0)
    def _init():
        ckb_s[...] = jnp.concatenate(
            [ctk_ref[j].astype(bf16) for j in range(_ROW // 128)], axis=1)
        cnt_k = jnp.concatenate(
            [jnp.sum(ctk_ref[j], axis=0, keepdims=True)
             for j in range(_ROW // 128)], axis=1)
        rck_s[...] = 1.0 / jnp.maximum(cnt_k, 1.0)
        cgb_s[...] = jnp.concatenate(
            [cg_ref[j].astype(bf16) for j in range(_COL // 128)], axis=1)
        cnt_g = jnp.sum(cg_ref[0], axis=1, keepdims=True)
        for j in range(1, _COL // 128):
            cnt_g += jnp.sum(cg_ref[j], axis=1, keepdims=True)
        rcg_s[...] = 1.0 / jnp.maximum(cnt_g, 1.0)
        e_s[...] = x_ref[...]

    e = e_s[...]
    eb = e.astype(bf16)
    wlcb = wlc_ref[0]
    wrcb = wrc_ref[0]
    t_raw = jnp.dot(eb, ckb_s[...], preferred_element_type=f32)
    t = (t_raw * rck_s[...]).astype(bf16)
    h = jnp.dot(wrcb, eb, preferred_element_type=f32)
    h += jnp.dot(wlcb, t, preferred_element_type=f32)
    h += bc_ref[0]
    e1 = jnp.where(h >= 0, h, h * 0.01)
    e1b = e1.astype(bf16)
    wlrb = wlrT_ref[0]
    wrrb = wrrT_ref[0]
    u_raw = jnp.dot(cgb_s[...], e1b, preferred_element_type=f32)
    u = (u_raw * rcg_s[...]).astype(bf16)
    h2 = jnp.dot(e1b, wrrb, preferred_element_type=f32)
    h2 += jnp.dot(u, wlrb, preferred_element_type=f32)
    h2 += br_ref[0]
    e2 = jnp.where(h2 >= 0, h2, h2 * 0.01)
    e_s[...] = e2

    @pl.when(i == _LAYERS - 1)
    def _fin():
        out_ref[...] = e2


def _tc_forward(ctk, cg, x, Wl_c, Wr_c, bc, wlrT, wrrT, br):
    f32 = jnp.float32
    return pl.pallas_call(
        _tc_net_body,
        grid=(_LAYERS,),
        in_specs=[
            pl.BlockSpec((_ROW // 128, _ROW, 128), lambda i: (0, 0, 0)),
            pl.BlockSpec((_COL // 128, _COL, 128), lambda i: (0, 0, 0)),
            pl.BlockSpec((_COL, _ROW), lambda i: (0, 0)),
            pl.BlockSpec((1, _COL, _COL), lambda i: (i, 0, 0)),
            pl.BlockSpec((1, _COL, _COL), lambda i: (i, 0, 0)),
            pl.BlockSpec((1, _COL, 1), lambda i: (i, 0, 0)),
            pl.BlockSpec((1, _ROW, _ROW), lambda i: (i, 0, 0)),
            pl.BlockSpec((1, _ROW, _ROW), lambda i: (i, 0, 0)),
            pl.BlockSpec((1, 1, _ROW), lambda i: (i, 0, 0)),
        ],
        out_specs=pl.BlockSpec((_COL, _ROW), lambda i: (0, 0)),
        out_shape=jax.ShapeDtypeStruct((_COL, _ROW), f32),
        scratch_shapes=[
            pltpu.VMEM((_COL, _ROW), f32),
            pltpu.VMEM((_ROW, _ROW), jnp.bfloat16),
            pltpu.VMEM((_COL, _COL), jnp.bfloat16),
            pltpu.VMEM((1, _ROW), f32),
            pltpu.VMEM((_COL, 1), f32),
        ],
        compiler_params=pltpu.CompilerParams(
            dimension_semantics=("arbitrary",),
        ),
    )(ctk, cg, x, Wl_c, Wr_c, bc, wlrT, wrrT, br)


def kernel(x, knn_edge_index, genet_edge_index, Wl_c, Wr_c, b_c,
           Wl_r, Wr_r, b_r):
    ckf, cgf = _sc_counts(knn_edge_index, genet_edge_index)
    ctk = ckf.reshape(_ROW // 128, _ROW, 128)
    cg = cgf.reshape(_COL // 128, _COL, 128)
    wlcb, wrcb, wlrT, wrrT = _convert_weights(Wl_c, Wr_c, Wl_r, Wr_r)
    bc = b_c[:, :, None]
    br = b_r[:, None, :]
    return _tc_forward(ctk, cg, x, wlcb, wrcb, bc, wlrT, wrrT, br)


# speedup vs baseline: 1.0535x; 1.0535x over previous
import functools

import jax
import jax.numpy as jnp
from jax import lax
from jax.experimental import pallas as pl
from jax.experimental.pallas import tpu as pltpu
from jax.experimental.pallas import tpu_sc as plsc

_COL = 1024
_ROW = 512
_LAYERS = 3
_EK = 16384
_EG = 65536

_NC = 2
_NS = 16
_NW = _NC * _NS

_ZB = 4096


_SHK = (_ROW // _NC) * _ROW
_SHG = (_COL // _NC) * _COL
_TRASH = 128


def _sc_count_body(kei, gei, out_k, out_g, sh_k, sh_g,
                   zbuf, ones_v, idx_a, idx_b, idx_c, idx_d,
                   dvm_k, svm_k, dvm_g, svm_g,
                   sem_e, sem_z, sem_a, sem_b, sem_c, sem_d):
    c = lax.axis_index("c")
    s = lax.axis_index("s")

    ek = _EK // _NS
    eg = _EG // _NS
    loads = [
        pltpu.async_copy(kei.at[1, pl.ds(s * ek, ek)], dvm_k, sem_e),
        pltpu.async_copy(kei.at[0, pl.ds(s * ek, ek)], svm_k, sem_e),
        pltpu.async_copy(gei.at[1, pl.ds(s * eg, eg)], dvm_g, sem_e),
        pltpu.async_copy(gei.at[0, pl.ds(s * eg, eg)], svm_g, sem_e),
    ]
    zero16 = jnp.zeros((16,), jnp.float32)
    one16 = jnp.ones((16,), jnp.float32)

    def zfill(i, carry):
        zbuf[pl.ds(i * 16, 16)] = zero16
        return carry

    lax.fori_loop(0, _ZB // 16, zfill, 0)
    for j in range(8):
        ones_v[pl.ds(j * 16, 16)] = one16

    ksl = _SHK // _NS
    gsl = _SHG // _NS
    zeros = []
    for r in range(ksl // _ZB):
        zeros.append(pltpu.async_copy(
            zbuf, sh_k.at[pl.ds(s * ksl + r * _ZB, _ZB)], sem_z))
    for r in range(gsl // _ZB):
        zeros.append(pltpu.async_copy(
            zbuf, sh_g.at[pl.ds(s * gsl + r * _ZB, _ZB)], sem_z))
    for h in loads:
        h.wait()
    for h in zeros:
        h.wait()
    plsc.subcore_barrier()

    def scatter_graph(rvm, cvm, shared, n, e_tile, trash, blk_sh):
        r0 = c * (n // _NC)
        bufs = (idx_a, idx_b, idx_c, idx_d)
        sems = (sem_a, sem_b, sem_c, sem_d)
        handles = [None, None, None, None]
        for chunk in range(e_tile // 128):
            p = chunk % 4
            if handles[p] is not None:
                handles[p].wait()
            for j in range(8):
                off = chunk * 128 + j * 16
                rl = rvm[pl.ds(off, 16)] - r0
                cc = cvm[pl.ds(off, 16)]
                ok = (rl >= 0) & (rl < (n // _NC))
                flat = ((cc >> 7) << blk_sh) | (rl << 7) | (cc & 127)
                flat = jnp.where(ok, flat, trash + (cc & 127))
                bufs[p][pl.ds(j * 16, 16)] = flat
            handles[p] = pltpu.async_copy(
                ones_v, shared.at[bufs[p]], sems[p], add=True)
        for h in handles:
            if h is not None:
                h.wait()

    scatter_graph(svm_k, dvm_k, sh_k, _ROW, ek, _SHK, 15)
    scatter_graph(dvm_g, svm_g, sh_g, _COL, eg, _SHG, 16)
    plsc.subcore_barrier()

    outs = []
    kb = _SHK // 4
    gb = _SHG // 8
    for j in range(4):
        outs.append(pltpu.async_copy(
            sh_k.at[pl.ds(j * kb + s * (kb // _NS), kb // _NS)],
            out_k.at[pl.ds(j * 2 * kb + c * kb + s * (kb // _NS), kb // _NS)],
            sem_a))
    for j in range(8):
        outs.append(pltpu.async_copy(
            sh_g.at[pl.ds(j * gb + s * (gb // _NS), gb // _NS)],
            out_g.at[pl.ds(j * 2 * gb + c * gb + s * (gb // _NS), gb // _NS)],
            sem_b))
    for h in outs:
        h.wait()


def _sc_counts(kei, gei):
    mesh = plsc.VectorSubcoreMesh(core_axis_name="c", subcore_axis_name="s")
    f32 = jnp.float32
    i32 = jnp.int32
    run = functools.partial(
        pl.kernel,
        mesh=mesh,
        out_type=[
            jax.ShapeDtypeStruct((_ROW * _ROW,), f32),
            jax.ShapeDtypeStruct((_COL * _COL,), f32),
        ],
        scratch_types=[
            pltpu.VMEM_SHARED((_SHK + _TRASH,), f32),
            pltpu.VMEM_SHARED((_SHG + _TRASH,), f32),
            pltpu.VMEM((_ZB,), f32),
            pltpu.VMEM((128,), f32),
            pltpu.VMEM((128,), i32),
            pltpu.VMEM((128,), i32),
            pltpu.VMEM((128,), i32),
            pltpu.VMEM((128,), i32),
            pltpu.VMEM((_EK // _NS,), i32),
            pltpu.VMEM((_EK // _NS,), i32),
            pltpu.VMEM((_EG // _NS,), i32),
            pltpu.VMEM((_EG // _NS,), i32),
            pltpu.SemaphoreType.DMA,
            pltpu.SemaphoreType.DMA,
            pltpu.SemaphoreType.DMA,
            pltpu.SemaphoreType.DMA,
            pltpu.SemaphoreType.DMA,
            pltpu.SemaphoreType.DMA,
        ],
    )(_sc_count_body)
    return run(kei, gei)


def _tc_net_body(ctk_ref, cg_ref, x_ref, wlc_ref, wrc_ref, bc_ref,
                 wlrT_ref, wrrT_ref, br_ref, out_ref,
                 e_s, ckb_s, cgb_s, rck_s, rcg_s):
    i = pl.program_id(0)
    f32 = jnp.float32
    bf16 = jnp.bfloat16

    @pl.when(i == 0)
    def _init():
        ckb_s[...] = jnp.concatenate(
            [ctk_ref[j].astype(bf16) for j in range(_ROW // 128)], axis=1)
        cnt_k = jnp.concatenate(
            [jnp.sum(ctk_ref[j], axis=0, keepdims=True)
             for j in range(_ROW // 128)], axis=1)
        rck_s[...] = 1.0 / jnp.maximum(cnt_k, 1.0)
        cgb_s[...] = jnp.concatenate(
            [cg_ref[j].astype(bf16) for j in range(_COL // 128)], axis=1)
        cnt_g = jnp.sum(cg_ref[0], axis=1, keepdims=True)
        for j in range(1, _COL // 128):
            cnt_g += jnp.sum(cg_ref[j], axis=1, keepdims=True)
        rcg_s[...] = 1.0 / jnp.maximum(cnt_g, 1.0)
        e_s[...] = x_ref[...]

    e = e_s[...]
    eb = e.astype(bf16)
    wlcb = wlc_ref[0].astype(bf16)
    wrcb = wrc_ref[0].astype(bf16)
    t_raw = jnp.dot(eb, ckb_s[...], preferred_element_type=f32)
    t = (t_raw * rck_s[...]).astype(bf16)
    h = jnp.dot(wrcb, eb, preferred_element_type=f32)
    h += jnp.dot(wlcb, t, preferred_element_type=f32)
    h += bc_ref[pl.ds(i, 1), :].T
    e1 = jnp.where(h >= 0, h, h * 0.01)
    e1b = e1.astype(bf16)
    wlrb = wlrT_ref[0].astype(bf16)
    wrrb = wrrT_ref[0].astype(bf16)
    u_raw = jnp.dot(cgb_s[...], e1b, preferred_element_type=f32)
    u = (u_raw * rcg_s[...]).astype(bf16)
    h2 = jnp.dot(e1b, wrrb, preferred_element_type=f32)
    h2 += jnp.dot(u, wlrb, preferred_element_type=f32)
    h2 += br_ref[pl.ds(i, 1), :]
    e2 = jnp.where(h2 >= 0, h2, h2 * 0.01)
    e_s[...] = e2

    @pl.when(i == _LAYERS - 1)
    def _fin():
        out_ref[...] = e2


def _tc_forward(ctk, cg, x, Wl_c, Wr_c, bc, wlrT, wrrT, br):
    f32 = jnp.float32
    return pl.pallas_call(
        _tc_net_body,
        grid=(_LAYERS,),
        in_specs=[
            pl.BlockSpec((_ROW // 128, _ROW, 128), lambda i: (0, 0, 0)),
            pl.BlockSpec((_COL // 128, _COL, 128), lambda i: (0, 0, 0)),
            pl.BlockSpec((_COL, _ROW), lambda i: (0, 0)),
            pl.BlockSpec((1, _COL, _COL), lambda i: (i, 0, 0)),
            pl.BlockSpec((1, _COL, _COL), lambda i: (i, 0, 0)),
            pl.BlockSpec((_LAYERS, _COL), lambda i: (0, 0)),
            pl.BlockSpec((1, _ROW, _ROW), lambda i: (i, 0, 0)),
            pl.BlockSpec((1, _ROW, _ROW), lambda i: (i, 0, 0)),
            pl.BlockSpec((_LAYERS, _ROW), lambda i: (0, 0)),
        ],
        out_specs=pl.BlockSpec((_COL, _ROW), lambda i: (0, 0)),
        out_shape=jax.ShapeDtypeStruct((_COL, _ROW), f32),
        scratch_shapes=[
            pltpu.VMEM((_COL, _ROW), f32),
            pltpu.VMEM((_ROW, _ROW), jnp.bfloat16),
            pltpu.VMEM((_COL, _COL), jnp.bfloat16),
            pltpu.VMEM((1, _ROW), f32),
            pltpu.VMEM((_COL, 1), f32),
        ],
        compiler_params=pltpu.CompilerParams(
            dimension_semantics=("arbitrary",),
        ),
    )(ctk, cg, x, Wl_c, Wr_c, bc, wlrT, wrrT, br)


def kernel(x, knn_edge_index, genet_edge_index, Wl_c, Wr_c, b_c,
           Wl_r, Wr_r, b_r):
    ckf, cgf = _sc_counts(knn_edge_index, genet_edge_index)
    ctk = ckf.reshape(_ROW // 128, _ROW, 128)
    cg = cgf.reshape(_COL // 128, _COL, 128)
    wlrT = Wl_r.transpose(0, 2, 1)
    wrrT = Wr_r.transpose(0, 2, 1)
    return _tc_forward(ctk, cg, x, Wl_c, Wr_c, b_c, wlrT, wrrT, b_r)
